# Initial kernel scaffold; baseline (speedup 1.0000x reference)
import jax
import jax.numpy as jnp
from jax.experimental import pallas as pl


def kernel(x, input_ids, lookup_table, multipliers, head_sizes, offsets, table, W_o):
    raise NotImplementedError("write your pallas kernel here")


# speedup vs baseline: 9.1683x; 9.1683x over previous
import functools

import numpy as np
import jax
import jax.numpy as jnp
from jax import lax
from jax.experimental import pallas as pl
from jax.experimental.pallas import tpu as pltpu
from jax.experimental.pallas import tpu_sc as plsc

NC = 2
NS = 16
L = 16
NW = NC * NS

MAXN = 3
NHEAD_TOT = 4
DHEAD = 512

_MASK16 = 0xFFFF


def _srl16(v):
    return lax.shift_right_logical(v, jnp.asarray(16, v.dtype))


def _z():
    return jnp.int32(0)


def _sc_gather_kernel(T, n_rows):
    TPT = T // NW
    CH = 64
    NITER = TPT // L

    mesh = plsc.VectorSubcoreMesh(core_axis_name="c", subcore_axis_name="s")

    @functools.partial(
        pl.kernel,
        mesh=mesh,
        out_type=jax.ShapeDtypeStruct((NHEAD_TOT, T, DHEAD), jnp.float32),
        scratch_types=[
            pltpu.VMEM((MAXN, TPT), jnp.int32),
            pltpu.VMEM((22, L), jnp.int32),
            pltpu.VMEM((NHEAD_TOT, L), jnp.float32),
            pltpu.VMEM((NHEAD_TOT, TPT), jnp.int32),
            pltpu.VMEM((2, CH, DHEAD), jnp.float32),
            pltpu.SemaphoreType.DMA,
            pltpu.SemaphoreType.DMA,
        ],
    )
    def k(sh_hbm, ci_hbm, cf_hbm, table_hbm, emb_hbm,
          sh_v, ci_v, cf_v, idx_v, rows_v, sem0, sem1):
        wid = lax.axis_index("s") * NC + lax.axis_index("c")
        base = wid * TPT

        pltpu.sync_copy(sh_hbm.at[:, pl.ds(base, TPT)], sh_v)
        pltpu.sync_copy(ci_hbm, ci_v)
        pltpu.sync_copy(cf_hbm, cf_v)

        a = [ci_v[k, :] for k in range(MAXN)]
        b = [ci_v[MAXN + k, :] for k in range(MAXN)]
        p = [ci_v[6 + h, :] for h in range(NHEAD_TOT)]
        c1 = [ci_v[10 + h, :] for h in range(NHEAD_TOT)]
        c2 = [ci_v[14 + h, :] for h in range(NHEAD_TOT)]
        off = [ci_v[18 + h, :] for h in range(NHEAD_TOT)]
        pinv = [cf_v[h, :] for h in range(NHEAD_TOT)]

        def limbs(xk, k):
            u = xk * a[k]
            v = xk * b[k]
            u_lo = jnp.bitwise_and(u, _MASK16)
            u_hi = _srl16(u)
            v_lo = jnp.bitwise_and(v, _MASK16)
            v_hi = _srl16(v)
            t = u_lo + v_hi
            return (v_lo,
                    jnp.bitwise_and(t, _MASK16),
                    u_hi + _srl16(t))

        def mod_head(L0, L1, L2, h):
            L2p = jnp.where(L2 >= p[h], L2 - p[h], L2)
            L1p = jnp.where(L1 >= p[h], L1 - p[h], L1)
            acc1 = L2p * c2[h]
            acc = (_srl16(acc1) * c1[h]
                   + jnp.bitwise_and(acc1, _MASK16)
                   + L1p * c1[h] + L0)
            q = (acc.astype(jnp.float32) * pinv[h]).astype(jnp.int32)
            r = acc - q * p[h]
            r = jnp.where(r < 0, r + p[h], r)
            r = jnp.where(r < 0, r + p[h], r)
            r = jnp.where(r >= p[h], r - p[h], r)
            r = jnp.where(r >= p[h], r - p[h], r)
            return r + off[h]

        for i in range(NITER):
            ds_i = pl.ds(i * L, L)
            s0 = sh_v[0, ds_i]
            s1 = sh_v[1, ds_i]
            s2 = sh_v[2, ds_i]
            P0 = limbs(s0, 0)
            P1 = limbs(s1, 1)
            P2 = limbs(s2, 2)
            m2 = tuple(jnp.bitwise_xor(P0[j], P1[j]) for j in range(3))
            m3 = tuple(jnp.bitwise_xor(m2[j], P2[j]) for j in range(3))
            idx_v[0, ds_i] = mod_head(m2[0], m2[1], m2[2], 0)
            idx_v[1, ds_i] = mod_head(m2[0], m2[1], m2[2], 1)
            idx_v[2, ds_i] = mod_head(m3[0], m3[1], m3[2], 2)
            idx_v[3, ds_i] = mod_head(m3[0], m3[1], m3[2], 3)

        sems = [sem0, sem1]
        chunks = [(h, c) for h in range(NHEAD_TOT) for c in range(TPT // CH)]
        copies = [None, None]
        for j in range(len(chunks) + 1):
            if j < len(chunks):
                h, c = chunks[j]
                buf = j % 2
                copies[buf] = pltpu.async_copy(
                    table_hbm.at[idx_v.at[jnp.int32(h), pl.ds(c * CH, CH)]],
                    rows_v.at[jnp.int32(buf)], sems[buf])
            if j > 0:
                hp, cp = chunks[j - 1]
                bufp = (j - 1) % 2
                copies[bufp].wait()
                pltpu.sync_copy(
                    rows_v.at[jnp.int32(bufp)],
                    emb_hbm.at[jnp.int32(hp), pl.ds(base + cp * CH, CH)])

    return k


def _tc_kernel(BT, D):
    def body(x_ref, emb_ref, w_ref, out_ref):
        xb = x_ref[:]
        ms = jnp.mean(xb * xb, axis=-1, keepdims=True) + 1e-6
        xn = xb * lax.rsqrt(ms)
        e = emb_ref[:]
        s = (jnp.sum(xn[:, :DHEAD] * e[0], axis=-1)
             + jnp.sum(xn[:, DHEAD:] * e[1], axis=-1))
        gate = 1.0 / (1.0 + jnp.exp(-s * (1.0 / jnp.sqrt(jnp.float32(D)))))
        val = jnp.concatenate([e[2], e[3]], axis=-1)
        gv = gate[:, None] * val
        out_ref[:] = lax.dot_general(
            gv, w_ref[:], (((1,), (1,)), ((), ())),
            preferred_element_type=jnp.float32)
    return body


def kernel(x, input_ids, lookup_table, multipliers, head_sizes, offsets,
           table, W_o):
    B, S, D = x.shape
    T = B * S

    xc = jnp.take(lookup_table, input_ids, axis=0)
    xc32 = xc.astype(jnp.int32)
    z = jnp.zeros((B, 1), jnp.int32)
    sh1 = jnp.concatenate([z, xc32[:, :-1]], axis=1)
    sh2 = jnp.concatenate([z, z, xc32[:, :-2]], axis=1)
    sh = jnp.stack([xc32.reshape(T), sh1.reshape(T), sh2.reshape(T)], axis=0)

    m = multipliers.astype(jnp.int64)
    a16 = _srl16(m).astype(jnp.int32)
    b16 = jnp.bitwise_and(m, _MASK16).astype(jnp.int32)
    ps64 = head_sizes.astype(jnp.int64)
    c1 = jnp.remainder(jnp.int64(1 << 16), ps64).astype(jnp.int32)
    c2 = jnp.remainder(jnp.int64(1 << 32), ps64).astype(jnp.int32)
    p32 = head_sizes.astype(jnp.int32)
    off32 = offsets.astype(jnp.int32)
    ci = jnp.concatenate([a16, b16, p32, c1, c2, off32])
    ci_b = jnp.broadcast_to(ci[:, None], (22, L)).astype(jnp.int32)
    pinv = (1.0 / ps64.astype(jnp.float32))
    cf_b = jnp.broadcast_to(pinv[:, None], (NHEAD_TOT, L))

    emb = _sc_gather_kernel(T, table.shape[0])(sh, ci_b, cf_b, table)

    BT = 512
    x2d = x.reshape(T, D)
    out = pl.pallas_call(
        _tc_kernel(BT, D),
        grid=(T // BT,),
        in_specs=[
            pl.BlockSpec((BT, D), lambda i: (i, _z())),
            pl.BlockSpec((NHEAD_TOT, BT, DHEAD), lambda i: (_z(), i, _z())),
            pl.BlockSpec((D, D), lambda i: (_z(), _z())),
        ],
        out_specs=pl.BlockSpec((BT, D), lambda i: (i, _z())),
        out_shape=jax.ShapeDtypeStruct((T, D), jnp.float32),
    )(x2d, emb, W_o.astype(jnp.float32))

    return out.reshape(B, S, D).astype(jnp.float64)
